# Initial kernel scaffold; baseline (speedup 1.0000x reference)
import functools

import jax
import jax.numpy as jnp
from jax import lax
from jax.experimental import pallas as pl
from jax.experimental.pallas import tpu as pltpu
from jax.experimental.pallas import tpu_sc as plsc

NC = 2
NS = 16
NW = NC * NS

VOCAB = 100000
D = 64
B_TOTAL = 4096 * 50
BPW = B_TOTAL // NW
CHUNK = 640
NCHUNK = BPW // CHUNK


def _make_gather():
  mesh = plsc.VectorSubcoreMesh(
      core_axis_name="c", subcore_axis_name="s",
      num_cores=NC, num_subcores=NS)

  @functools.partial(
      pl.kernel,
      mesh=mesh,
      out_type=jax.ShapeDtypeStruct((B_TOTAL, D), jnp.float32),
      scratch_types=[
          pltpu.VMEM((BPW,), jnp.int32),
          pltpu.VMEM((CHUNK, D), jnp.float32),
          pltpu.SemaphoreType.DMA,
      ],
  )
  def gather_kernel(table_hbm, idx_hbm, out_hbm, idx_v, rows_v, sem):
    wid = lax.axis_index("s") * NC + lax.axis_index("c")
    base = wid * BPW
    pltpu.sync_copy(idx_hbm.at[pl.ds(base, BPW)], idx_v)
    for j in range(NCHUNK):
      pltpu.async_copy(
          table_hbm.at[idx_v.at[pl.ds(j * CHUNK, CHUNK)]], rows_v, sem
      ).wait()
      pltpu.sync_copy(rows_v, out_hbm.at[pl.ds(base + j * CHUNK, CHUNK)])

  return gather_kernel


_gather = _make_gather()


@jax.jit
def kernel(shifts, idx):
  b, h = idx.shape
  flat = idx.reshape(b * h)
  out = _gather(shifts, flat)
  return out.reshape(b, h, D)


# speedup vs baseline: 4.5694x; 4.5694x over previous
import functools

import jax
import jax.numpy as jnp
from jax import lax
from jax.experimental import pallas as pl
from jax.experimental.pallas import tpu as pltpu
from jax.experimental.pallas import tpu_sc as plsc

NC = 2
NS = 16
NW = NC * NS

VOCAB = 100000
D = 64
B_TOTAL = 4096 * 50
BPW = B_TOTAL // NW
CHUNK = 640
NCHUNK = BPW // CHUNK


def _make_gather():
  mesh = plsc.VectorSubcoreMesh(
      core_axis_name="c", subcore_axis_name="s",
      num_cores=NC, num_subcores=NS)

  @functools.partial(
      pl.kernel,
      mesh=mesh,
      out_type=jax.ShapeDtypeStruct((B_TOTAL, D), jnp.float32),
      scratch_types=[
          pltpu.VMEM((BPW,), jnp.int32),
          pltpu.VMEM((CHUNK, D), jnp.float32),
          pltpu.SemaphoreType.DMA,
      ],
      compiler_params=pltpu.CompilerParams(use_tc_tiling_on_sc=False),
  )
  def gather_kernel(table_hbm, idx_hbm, out_hbm, idx_v, rows_v, sem):
    wid = lax.axis_index("s") * NC + lax.axis_index("c")
    base = wid * BPW
    pltpu.sync_copy(idx_hbm.at[pl.ds(base, BPW)], idx_v)
    for j in range(NCHUNK):
      pltpu.async_copy(
          table_hbm.at[idx_v.at[pl.ds(j * CHUNK, CHUNK)]], rows_v, sem
      ).wait()
      pltpu.sync_copy(rows_v, out_hbm.at[pl.ds(base + j * CHUNK, CHUNK)])

  return gather_kernel


_gather = _make_gather()


@jax.jit
def kernel(shifts, idx):
  b, h = idx.shape
  flat = idx.reshape(b * h)
  out = _gather(shifts, flat)
  return out.reshape(b, h, D)


# speedup vs baseline: 4.6677x; 1.0215x over previous
import functools

import jax
import jax.numpy as jnp
from jax import lax
from jax.experimental import pallas as pl
from jax.experimental.pallas import tpu as pltpu
from jax.experimental.pallas import tpu_sc as plsc

NC = 2
NS = 16
NW = NC * NS

VOCAB = 100000
D = 64
B_TOTAL = 4096 * 50
BPW = B_TOTAL // NW
CHUNK = 640
NCHUNK = BPW // CHUNK


def _make_gather():
  mesh = plsc.VectorSubcoreMesh(
      core_axis_name="c", subcore_axis_name="s",
      num_cores=NC, num_subcores=NS)

  @functools.partial(
      pl.kernel,
      mesh=mesh,
      out_type=jax.ShapeDtypeStruct((B_TOTAL, D), jnp.float32),
      scratch_types=[
          pltpu.VMEM((BPW,), jnp.int32),
          pltpu.VMEM((CHUNK, D), jnp.float32),
          pltpu.VMEM((CHUNK, D), jnp.float32),
          pltpu.SemaphoreType.DMA,
          pltpu.SemaphoreType.DMA,
          pltpu.SemaphoreType.DMA,
          pltpu.SemaphoreType.DMA,
      ],
      compiler_params=pltpu.CompilerParams(use_tc_tiling_on_sc=False),
  )
  def gather_kernel(table_hbm, idx_hbm, out_hbm, idx_v, rows0, rows1,
                    gsem0, gsem1, osem0, osem1):
    wid = lax.axis_index("s") * NC + lax.axis_index("c")
    base = wid * BPW
    bufs = (rows0, rows1)
    gsems = (gsem0, gsem1)
    osems = (osem0, osem1)
    pltpu.sync_copy(idx_hbm.at[pl.ds(base, BPW)], idx_v)

    def start_gather(j):
      return pltpu.async_copy(
          table_hbm.at[idx_v.at[pl.ds(j * CHUNK, CHUNK)]],
          bufs[j % 2], gsems[j % 2])

    gd = [None, None]
    od = [None, None]
    gd[0] = start_gather(0)
    for j in range(NCHUNK):
      nxt = j + 1
      if nxt < NCHUNK:
        if od[nxt % 2] is not None:
          od[nxt % 2].wait()
        gd[nxt % 2] = start_gather(nxt)
      gd[j % 2].wait()
      od[j % 2] = pltpu.async_copy(
          bufs[j % 2], out_hbm.at[pl.ds(base + j * CHUNK, CHUNK)],
          osems[j % 2])
    od[0].wait()
    od[1].wait()

  return gather_kernel


_gather = _make_gather()


@jax.jit
def kernel(shifts, idx):
  b, h = idx.shape
  flat = idx.reshape(b * h)
  out = _gather(shifts, flat)
  return out.reshape(b, h, D)


# speedup vs baseline: 4.6751x; 1.0016x over previous
import functools

import jax
import jax.numpy as jnp
from jax import lax
from jax.experimental import layout as jax_layout
from jax.experimental import pallas as pl
from jax.experimental.pallas import tpu as pltpu
from jax.experimental.pallas import tpu_sc as plsc

NC = 2
NS = 16
NW = NC * NS

VOCAB = 100000
D = 64
B_TOTAL = 4096 * 50
BPW = B_TOTAL // NW
CHUNK = 640
NCHUNK = BPW // CHUNK


def _make_gather():
  mesh = plsc.VectorSubcoreMesh(
      core_axis_name="c", subcore_axis_name="s",
      num_cores=NC, num_subcores=NS)

  @functools.partial(
      pl.kernel,
      mesh=mesh,
      out_type=jax.ShapeDtypeStruct((B_TOTAL, D), jnp.float32),
      scratch_types=[
          pltpu.VMEM((BPW,), jnp.int32),
          pltpu.VMEM((CHUNK, D), jnp.float32),
          pltpu.VMEM((CHUNK, D), jnp.float32),
          pltpu.SemaphoreType.DMA,
          pltpu.SemaphoreType.DMA,
          pltpu.SemaphoreType.DMA,
          pltpu.SemaphoreType.DMA,
      ],
      compiler_params=pltpu.CompilerParams(use_tc_tiling_on_sc=False),
  )
  def gather_kernel(table_hbm, idx_hbm, out_hbm, idx_v, rows0, rows1,
                    gsem0, gsem1, osem0, osem1):
    wid = lax.axis_index("s") * NC + lax.axis_index("c")
    base = wid * BPW
    bufs = (rows0, rows1)
    gsems = (gsem0, gsem1)
    osems = (osem0, osem1)
    pltpu.sync_copy(idx_hbm.at[pl.ds(base, BPW)], idx_v)

    def start_gather(j):
      return pltpu.async_copy(
          table_hbm.at[idx_v.at[pl.ds(j * CHUNK, CHUNK)]],
          bufs[j % 2], gsems[j % 2])

    gd = [None, None]
    od = [None, None]
    gd[0] = start_gather(0)
    for j in range(NCHUNK):
      nxt = j + 1
      if nxt < NCHUNK:
        if od[nxt % 2] is not None:
          od[nxt % 2].wait()
        gd[nxt % 2] = start_gather(nxt)
      gd[j % 2].wait()
      od[j % 2] = pltpu.async_copy(
          bufs[j % 2], out_hbm.at[pl.ds(base + j * CHUNK, CHUNK)],
          osems[j % 2])
    od[0].wait()
    od[1].wait()

  return gather_kernel


_gather = _make_gather()


def _kernel_impl(shifts, idx):
  b, h = idx.shape
  flat = idx.reshape(b * h)
  out = _gather(shifts, flat)
  return out.reshape(b, h, D)


_jitted_cache = {}


def kernel(shifts, idx):
  try:
    dev = next(iter(shifts.devices()))
  except Exception:
    dev = jax.devices()[0]
  fn = _jitted_cache.get(dev)
  if fn is None:
    fmt = jax_layout.Format(
        jax_layout.Layout(major_to_minor=(0, 1, 2)),
        jax.sharding.SingleDeviceSharding(dev))
    fn = jax.jit(_kernel_impl, out_shardings=fmt)
    _jitted_cache[dev] = fn
  return fn(shifts, idx)
